# Initial kernel scaffold; baseline (speedup 1.0000x reference)
import jax
import jax.numpy as jnp
from jax.experimental import pallas as pl


def kernel(x, weight, bias, gamma, beta):
    raise NotImplementedError("write your pallas kernel here")


# speedup vs baseline: 1.1966x; 1.1966x over previous
import jax
import jax.numpy as jnp
from jax.experimental import pallas as pl
from jax.experimental.pallas import tpu as pltpu

NEG_SLOPE = 0.01
BN_EPS = 1e-5


def kernel(x, weight, bias, gamma, beta):
    del bias
    N, Cin, H, W = x.shape
    Cout, Cin2, K, K2 = weight.shape
    assert Cin2 == Cin and K == K2 == 3
    pad = K // 2
    assert H % 2 == 0 and W % 2 == 0
    Ho, Wo = H // 2, W // 2
    Hp = H + 2 * pad
    J = Wo + 1
    T = 4
    assert Ho % T == 0
    G = Ho // T
    LN = J * N
    NWO = Wo * N
    KKC = K * K * Cin
    L = T * NWO
    inv_count = 1.0 / float(N * Ho * Wo)

    xp = jnp.pad(x, ((0, 0), (0, 0), (pad, pad), (pad, pad)), mode="reflect")
    xe = xp[:, :, :, 0::2].transpose(1, 2, 3, 0).reshape(Cin, Hp, LN)
    xo = xp[:, :, :, 1::2].transpose(1, 2, 3, 0).reshape(Cin, Hp, LN)
    xr = jnp.concatenate([xe, xo], axis=2).astype(jnp.bfloat16)
    xh = jnp.stack(
        [xr[:, 2 * T * (g + 1):2 * T * (g + 1) + 2, :] for g in range(G)], axis=0
    )

    wmat = weight.transpose(0, 2, 3, 1).reshape(Cout, KKC).astype(jnp.bfloat16)

    def conv_pool_stats_kernel(x_ref, halo_ref, w_ref, pooled_ref, stats_ref):
        xwin = jnp.concatenate([x_ref[...], halo_ref[0]], axis=1)
        w = w_ref[...]

        pooled_rows = []
        for p in range(T):
            conv_rows = []
            for dh in range(2):
                hl = 2 * p + dh
                pieces = []
                for kh in range(K):
                    row = xwin[:, hl + kh, :]
                    pe0 = row[:, 0:NWO]
                    pe1 = row[:, N:N + NWO]
                    po0 = row[:, LN:LN + NWO]
                    po1 = row[:, LN + N:LN + N + NWO]
                    pieces.append(jnp.concatenate([pe0, po0], axis=1))
                    pieces.append(jnp.concatenate([po0, pe1], axis=1))
                    pieces.append(jnp.concatenate([pe1, po1], axis=1))
                patch = jnp.concatenate(pieces, axis=0)
                conv_rows.append(
                    jnp.dot(w, patch, preferred_element_type=jnp.float32)
                )
            ymax = jnp.maximum(conv_rows[0], conv_rows[1])
            pooled_rows.append(jnp.maximum(ymax[:, :NWO], ymax[:, NWO:]))
        pooled = jnp.concatenate(pooled_rows, axis=1)
        pooled_ref[0] = pooled

        s1 = jnp.sum(pooled, axis=1, keepdims=True)
        s2 = jnp.sum(pooled * pooled, axis=1, keepdims=True)
        stats_ref[0] = jnp.concatenate([s1, s2], axis=1)

    pooled_parts, stats_parts = pl.pallas_call(
        conv_pool_stats_kernel,
        out_shape=(
            jax.ShapeDtypeStruct((G, Cout, L), jnp.float32),
            jax.ShapeDtypeStruct((G, Cout, 2), jnp.float32),
        ),
        grid=(G,),
        in_specs=[
            pl.BlockSpec((Cin, 2 * T, 2 * LN), lambda g: (0, g, 0)),
            pl.BlockSpec((1, Cin, 2, 2 * LN), lambda g: (g, 0, 0, 0)),
            pl.BlockSpec((Cout, KKC), lambda g: (0, 0)),
        ],
        out_specs=(
            pl.BlockSpec((1, Cout, L), lambda g: (g, 0, 0)),
            pl.BlockSpec((1, Cout, 2), lambda g: (g, 0, 0)),
        ),
        compiler_params=pltpu.CompilerParams(dimension_semantics=("parallel",)),
    )(xr, xh, wmat)

    stats_tot = jnp.sum(stats_parts, axis=0)
    params = jnp.concatenate(
        [gamma.reshape(Cout, 1), beta.reshape(Cout, 1), stats_tot], axis=1
    ).astype(jnp.float32)

    def bn_act_kernel(pooled_ref, params_ref, out_ref):
        po = pooled_ref[0]
        prm = params_ref[...]
        gam, bet = prm[:, 0:1], prm[:, 1:2]
        mean = prm[:, 2:3] * inv_count
        var = prm[:, 3:4] * inv_count - mean * mean
        var = jnp.maximum(var, 0.0)
        scale = gam * jax.lax.rsqrt(var + BN_EPS)
        shift = bet - mean * scale
        z = po * scale + shift
        out_ref[0] = jnp.maximum(z, NEG_SLOPE * z)

    y_parts = pl.pallas_call(
        bn_act_kernel,
        out_shape=jax.ShapeDtypeStruct((G, Cout, L), jnp.float32),
        grid=(G,),
        in_specs=[
            pl.BlockSpec((1, Cout, L), lambda g: (g, 0, 0)),
            pl.BlockSpec((Cout, 4), lambda g: (0, 0)),
        ],
        out_specs=pl.BlockSpec((1, Cout, L), lambda g: (g, 0, 0)),
        compiler_params=pltpu.CompilerParams(dimension_semantics=("parallel",)),
    )(pooled_parts, params)

    out = (
        y_parts.reshape(G, Cout, T, Wo, N)
        .transpose(4, 1, 0, 2, 3)
        .reshape(N, Cout, Ho, Wo)
    )
    return out


# speedup vs baseline: 1.4393x; 1.2028x over previous
import jax
import jax.numpy as jnp
from jax.experimental import pallas as pl
from jax.experimental.pallas import tpu as pltpu

NEG_SLOPE = 0.01
BN_EPS = 1e-5


def kernel(x, weight, bias, gamma, beta):
    del bias
    N, Cin, H, W = x.shape
    Cout, Cin2, K, K2 = weight.shape
    assert Cin2 == Cin and K == K2 == 3
    pad = K // 2
    assert H % 2 == 0 and W % 2 == 0
    Ho, Wo = H // 2, W // 2
    Jh, Jw = Ho + 1, Wo + 1
    LQ = Jh * Jw
    LPAD = ((LQ + Jw + 1 + 127) // 128) * 128
    KKC = K * K * Cin
    NVALID = Ho * Wo
    inv_count = 1.0 / float(N * Ho * Wo)

    xp = jnp.pad(x, ((0, 0), (0, 0), (pad, pad), (pad, pad)), mode="reflect")
    quads = [
        xp[:, :, a::2, b::2].reshape(N, 1, Cin, LQ)
        for a in range(2) for b in range(2)
    ]
    xq = jnp.concatenate(quads, axis=1)
    xq = jnp.pad(xq, ((0, 0), (0, 0), (0, 0), (0, LPAD - LQ))).astype(jnp.bfloat16)

    wmat = weight.transpose(0, 2, 3, 1).reshape(Cout, KKC).astype(jnp.bfloat16)

    def conv_pool_stats_kernel(x_ref, w_ref, pooled_ref, stats_ref):
        q = x_ref[0]
        w = w_ref[...]

        quad_outs = []
        for a in range(2):
            for b in range(2):
                pieces = []
                for kh in range(K):
                    sa, di = (a + kh) % 2, (a + kh) // 2
                    for kw in range(K):
                        sb, dj = (b + kw) % 2, (b + kw) // 2
                        off = di * Jw + dj
                        pieces.append(q[sa * 2 + sb][:, off:off + LQ])
                patch = jnp.concatenate(pieces, axis=0)
                quad_outs.append(
                    jnp.dot(w, patch, preferred_element_type=jnp.float32)
                )
        pooled = jnp.maximum(
            jnp.maximum(quad_outs[0], quad_outs[1]),
            jnp.maximum(quad_outs[2], quad_outs[3]),
        )
        pooled_ref[0] = pooled

        li = jax.lax.broadcasted_iota(jnp.int32, (Cout, LQ), 1)
        valid = (li % Jw < Wo) & (li < Ho * Jw)
        pm = jnp.where(valid, pooled, 0.0)
        s1 = jnp.sum(pm, axis=1, keepdims=True)
        s2 = jnp.sum(pm * pm, axis=1, keepdims=True)
        stats_ref[0] = jnp.concatenate([s1, s2], axis=1)

    pooled_parts, stats_parts = pl.pallas_call(
        conv_pool_stats_kernel,
        out_shape=(
            jax.ShapeDtypeStruct((N, Cout, LQ), jnp.float32),
            jax.ShapeDtypeStruct((N, Cout, 2), jnp.float32),
        ),
        grid=(N,),
        in_specs=[
            pl.BlockSpec((1, 4, Cin, LPAD), lambda n: (n, 0, 0, 0)),
            pl.BlockSpec((Cout, KKC), lambda n: (0, 0)),
        ],
        out_specs=(
            pl.BlockSpec((1, Cout, LQ), lambda n: (n, 0, 0)),
            pl.BlockSpec((1, Cout, 2), lambda n: (n, 0, 0)),
        ),
        compiler_params=pltpu.CompilerParams(dimension_semantics=("parallel",)),
    )(xq, wmat)

    stats_tot = jnp.sum(stats_parts, axis=0)
    params = jnp.concatenate(
        [gamma.reshape(Cout, 1), beta.reshape(Cout, 1), stats_tot], axis=1
    ).astype(jnp.float32)

    def bn_act_kernel(pooled_ref, params_ref, out_ref):
        po = pooled_ref[0]
        prm = params_ref[...]
        gam, bet = prm[:, 0:1], prm[:, 1:2]
        mean = prm[:, 2:3] * inv_count
        var = prm[:, 3:4] * inv_count - mean * mean
        var = jnp.maximum(var, 0.0)
        scale = gam * jax.lax.rsqrt(var + BN_EPS)
        shift = bet - mean * scale
        z = po * scale + shift
        out_ref[0] = jnp.maximum(z, NEG_SLOPE * z)

    y_parts = pl.pallas_call(
        bn_act_kernel,
        out_shape=jax.ShapeDtypeStruct((N, Cout, LQ), jnp.float32),
        grid=(N,),
        in_specs=[
            pl.BlockSpec((1, Cout, LQ), lambda n: (n, 0, 0)),
            pl.BlockSpec((Cout, 4), lambda n: (0, 0)),
        ],
        out_specs=pl.BlockSpec((1, Cout, LQ), lambda n: (n, 0, 0)),
        compiler_params=pltpu.CompilerParams(dimension_semantics=("parallel",)),
    )(pooled_parts, params)

    out = y_parts.reshape(N, Cout, Jh, Jw)[:, :, :Ho, :Wo]
    return out
